# baseline (speedup 1.0000x reference)
import functools

import jax
import jax.numpy as jnp
from jax import lax
from jax.experimental import pallas as pl
from jax.experimental.pallas import tpu as pltpu
from jax.experimental.pallas import tpu_sc as plsc

SR = 137.8
P = 128
B = 4
BH = 2
N = 512
T = 4134
SIGMA = 3.6 / 1000.0 * SR
INV_SIG = 1.0 / SIGMA
W = 6
T_CAP = 4134.0

NC = 2
L = 16
NPG = 16
NP = N + L
NSUB = 1
SUBP = 8


def _take16(x, idx):
    dnums = lax.GatherDimensionNumbers(
        offset_dims=(), collapsed_slice_dims=(0,), start_index_map=(0,))
    return lax.gather(x, idx[:, None], dnums, (1,),
                      mode=lax.GatherScatterMode.PROMISE_IN_BOUNDS)


def _render_sc(sf, ef, vel, pit, zrows):
    mesh = plsc.VectorSubcoreMesh(core_axis_name="c", subcore_axis_name="s")

    @functools.partial(
        pl.kernel,
        mesh=mesh,
        out_type=jax.ShapeDtypeStruct((BH, 3 * P, T), jnp.float32),
        scratch_types=[
            pltpu.VMEM((NP,), jnp.float32),
            pltpu.VMEM((NP,), jnp.float32),
            pltpu.VMEM((NP,), jnp.float32),
            pltpu.VMEM((NP,), jnp.int32),
            pltpu.VMEM((NSUB * NP,), jnp.int32),
            pltpu.VMEM((SUBP, T), jnp.float32),
            pltpu.VMEM((SUBP, T), jnp.float32),
            pltpu.VMEM((SUBP, T), jnp.float32),
            pltpu.SemaphoreType.DMA,
            pltpu.SemaphoreType.DMA,
        ],
        compiler_params=pltpu.CompilerParams(use_tc_tiling_on_sc=True,
                                             needs_layout_passes=False),
    )
    def k(sf_hbm, ef_hbm, vel_hbm, pit_hbm, z_hbm, out_hbm,
          sfv, efv, velv, pitv, wl, bon, bsus, bvel, osem, zsem):
        cid = lax.axis_index("c")
        sid = lax.axis_index("s")
        wid = sid * NC + cid
        b = wid // NPG
        pg = wid % NPG
        plo = pg * SUBP

        bufs = (bon, bsus, bvel)

        zdescs = [pltpu.make_async_copy(z_hbm, bf, zsem) for bf in bufs]
        for cp in zdescs:
            cp.start()

        pltpu.sync_copy(sf_hbm.at[b], sfv)
        pltpu.sync_copy(ef_hbm.at[b], efv)
        pltpu.sync_copy(vel_hbm.at[b], velv)
        pltpu.sync_copy(pit_hbm.at[b], pitv)

        iota = lax.broadcasted_iota(jnp.int32, (L,), 0)

        scope = jax.named_scope
        ngrps = []
        for sub in range(NSUB):
            slo = plo + sub * SUBP

            def route(i, cnt, slo=slo, sub=sub):
                p16 = pitv[pl.ds(i * L, L)]
                sel = (p16 >= slo) & (p16 < slo + SUBP)
                pos = sub * NP + cnt + plsc.cumsum(jnp.where(sel, 1, 0)) - 1
                plsc.store_scatter(wl, [pos], i * L + iota, mask=sel)
                return cnt + plsc.all_reduce_population_count(sel)[0]
            with scope("route"):
                cnt = lax.fori_loop(0, N // L, route, 0)
            wl[pl.ds(sub * NP + cnt, L)] = jnp.full((L,), N, jnp.int32)
            ngrps.append((cnt + (L - 1)) // L)

        for sub in range(NSUB):
            slo = plo + sub * SUBP

            with scope("zwait"):
                for cp in zdescs:
                    cp.wait()

            def grp_body(g, _, slo=slo, sub=sub):
                ids = wl[pl.ds(sub * NP + g * L, L)]
                sfg = plsc.load_gather(sfv, [ids])
                efg = plsc.load_gather(efv, [ids])
                velg = plsc.load_gather(velv, [ids])
                pitg = plsc.load_gather(pitv, [ids])

                s0i = sfg.astype(jnp.int32)
                cfi = efg.astype(jnp.int32)
                ceii = cfi + jnp.where(cfi.astype(jnp.float32) < efg, 1, 0)
                s0g = s0i.astype(jnp.float32)
                e0g = jnp.minimum(ceii.astype(jnp.float32), T_CAP)
                lo16 = jnp.maximum(s0i - W, 0) & ~(L - 1)
                hi16 = jnp.minimum(ceii + (W + 1), T)
                row16 = pitg - slo

                def lane_body(j, _):
                    lane = jnp.zeros((L,), jnp.int32) + j
                    m = iota == j
                    sfb = _take16(sfg, lane)
                    efb = _take16(efg, lane)
                    velb = _take16(velg, lane)
                    s0b = _take16(s0g, lane)
                    e0b = _take16(e0g, lane)
                    loj = jnp.sum(jnp.where(m, lo16, 0))
                    hij = jnp.sum(jnp.where(m, hi16, 0))
                    rowj = jnp.sum(jnp.where(m, row16, 0))
                    ntrip = (jnp.maximum(hij - loj, 0) + (L - 1)) // L

                    def frame_body(kk, _):
                        t0 = loj + kk * L
                        tvi = t0 + iota
                        tvf = tvi.astype(jnp.float32)
                        zon = (tvf - sfb) * INV_SIG
                        on = jnp.exp(-0.5 * zon * zon) * velb
                        box = jnp.where((tvf >= s0b) & (tvf < e0b), 1.0, 0.0)
                        zof = (tvf - efb) * INV_SIG
                        g10 = jnp.where(tvf >= efb,
                                        jnp.exp(-0.5 * zof * zof) * 0.1, 0.0)
                        sus = box + g10
                        velc = sus * velb
                        plsc.addupdate(bon.at[rowj, pl.ds(t0, L)], on)
                        plsc.addupdate(bsus.at[rowj, pl.ds(t0, L)], sus)
                        plsc.addupdate(bvel.at[rowj, pl.ds(t0, L)], velc)
                        return 0

                    lax.fori_loop(0, ntrip, frame_body, 0)
                    return 0
                lax.fori_loop(0, L, lane_body, 0)
                return 0

            with scope("accum"):
                lax.fori_loop(0, ngrps[sub], grp_body, 0)

            cscope = scope("clip")
            cscope.__enter__()

            def clip_grp(g, _, sub=sub):
                ids = wl[pl.ds(sub * NP + g * L, L)]
                sfg = plsc.load_gather(sfv, [ids])
                efg = plsc.load_gather(efv, [ids])
                pitg = plsc.load_gather(pitv, [ids])
                s0i = sfg.astype(jnp.int32)
                cfi = efg.astype(jnp.int32)
                ceii = cfi + jnp.where(cfi.astype(jnp.float32) < efg, 1, 0)
                lo16 = jnp.maximum(s0i - W, 0) & ~(L - 1)
                hi16 = jnp.minimum(ceii + (W + 1), T)
                row16 = pitg - (plo + sub * SUBP)

                def clane(j, _):
                    m = iota == j
                    loj = jnp.sum(jnp.where(m, lo16, 0))
                    hij = jnp.sum(jnp.where(m, hi16, 0))
                    rowj = jnp.sum(jnp.where(m, row16, 0))
                    ntrip = (jnp.maximum(hij - loj, 0) + (L - 1)) // L

                    def cbody(kk, _):
                        sl = pl.ds(loj + kk * L, L)
                        bon[rowj, sl] = jnp.clip(bon[rowj, sl], 0.0, 1.0)
                        sl = pl.ds(loj + kk * L, L)
                        bsus[rowj, sl] = jnp.clip(bsus[rowj, sl], 0.0, 1.0)
                        sl = pl.ds(loj + kk * L, L)
                        bvel[rowj, sl] = jnp.clip(bvel[rowj, sl], 0.0, 1.0)
                        return 0
                    lax.fori_loop(0, ntrip, cbody, 0)
                    return 0
                lax.fori_loop(0, L, clane, 0)
                return 0
            lax.fori_loop(0, ngrps[sub], clip_grp, 0)

            cscope.__exit__(None, None, None)
            odescs = [
                pltpu.make_async_copy(bon, out_hbm.at[b, pl.ds(slo, SUBP)],
                                      osem),
                pltpu.make_async_copy(bsus, out_hbm.at[b, pl.ds(P + slo, SUBP)],
                                      osem),
                pltpu.make_async_copy(bvel, out_hbm.at[b, pl.ds(2 * P + slo, SUBP)],
                                      osem),
            ]
            for cp in odescs:
                cp.start()
            if sub + 1 < NSUB:
                zdescs = []
                with scope("owait"):
                    for ch in range(3):
                        odescs[ch].wait()
                        zcp = pltpu.make_async_copy(z_hbm, bufs[ch], zsem)
                        zcp.start()
                        zdescs.append(zcp)
            else:
                with scope("owait"):
                    for cp in odescs:
                        cp.wait()

    return k(sf, ef, vel, pit, zrows)


def kernel(note_start, note_end, note_vel, note_pitch, dur_sec):
    sr = jnp.float32(SR)
    sf = (note_start * sr).astype(jnp.float32)
    ef = (note_end * sr).astype(jnp.float32)
    vel = note_vel.astype(jnp.float32)
    pit = jnp.clip(note_pitch.astype(jnp.int32), 0, P - 1)
    pad = ((0, 0), (0, L))
    sf = jnp.pad(sf, pad, constant_values=-1e6)
    ef = jnp.pad(ef, pad, constant_values=-1e6)
    vel = jnp.pad(vel, pad, constant_values=0.0)
    pit = jnp.pad(pit, pad, constant_values=-1)
    zrows = jnp.zeros((SUBP, T), jnp.float32)
    o0 = _render_sc(sf[:BH], ef[:BH], vel[:BH], pit[:BH], zrows)
    o1 = _render_sc(sf[BH:], ef[BH:], vel[BH:], pit[BH:], zrows)
    return jnp.concatenate([o0, o1], axis=0)


# speedup vs baseline: 1.1358x; 1.1358x over previous
import functools

import jax
import jax.numpy as jnp
from jax import lax
from jax.experimental import pallas as pl
from jax.experimental.pallas import tpu as pltpu
from jax.experimental.pallas import tpu_sc as plsc

SR = 137.8
P = 128
B = 4
N = 512
T = 4134
SIGMA = 3.6 / 1000.0 * SR
INV_SIG = 1.0 / SIGMA
W = 6
T_CAP = 4134.0

NC = 2
L = 16
NPG = 8
NP = N + L
NSUB = 2
SUBP = 8


def _take16(x, idx):
    dnums = lax.GatherDimensionNumbers(
        offset_dims=(), collapsed_slice_dims=(0,), start_index_map=(0,))
    return lax.gather(x, idx[:, None], dnums, (1,),
                      mode=lax.GatherScatterMode.PROMISE_IN_BOUNDS)


def _render_sc(sf, ef, vel, pit, zrows):
    mesh = plsc.VectorSubcoreMesh(core_axis_name="c", subcore_axis_name="s")

    @functools.partial(
        pl.kernel,
        mesh=mesh,
        out_type=jax.ShapeDtypeStruct((B, 3 * P, T), jnp.float32),
        scratch_types=[
            pltpu.VMEM((NP,), jnp.float32),
            pltpu.VMEM((NP,), jnp.float32),
            pltpu.VMEM((NP,), jnp.float32),
            pltpu.VMEM((NP,), jnp.int32),
            pltpu.VMEM((NSUB * NP,), jnp.int32),
            pltpu.VMEM((SUBP, T), jnp.float32),
            pltpu.VMEM((SUBP, T), jnp.float32),
            pltpu.VMEM((SUBP, T), jnp.float32),
            pltpu.SemaphoreType.DMA,
            pltpu.SemaphoreType.DMA,
        ],
        compiler_params=pltpu.CompilerParams(use_tc_tiling_on_sc=True,
                                             needs_layout_passes=False),
    )
    def k(sf_hbm, ef_hbm, vel_hbm, pit_hbm, z_hbm, out_hbm,
          sfv, efv, velv, pitv, wl, bon, bsus, bvel, osem, zsem):
        cid = lax.axis_index("c")
        sid = lax.axis_index("s")
        wid = sid * NC + cid
        b = wid // NPG
        pg = wid % NPG
        plo = pg * 16

        bufs = (bon, bsus, bvel)

        zdescs = [pltpu.make_async_copy(z_hbm, bf, zsem) for bf in bufs]
        for cp in zdescs:
            cp.start()

        pltpu.sync_copy(sf_hbm.at[b], sfv)
        pltpu.sync_copy(ef_hbm.at[b], efv)
        pltpu.sync_copy(vel_hbm.at[b], velv)
        pltpu.sync_copy(pit_hbm.at[b], pitv)

        iota = lax.broadcasted_iota(jnp.int32, (L,), 0)

        ngrps = []
        for sub in range(NSUB):
            slo = plo + sub * SUBP

            def route(i, cnt, slo=slo, sub=sub):
                p16 = pitv[pl.ds(i * L, L)]
                sel = (p16 >= slo) & (p16 < slo + SUBP)
                pos = sub * NP + cnt + plsc.cumsum(jnp.where(sel, 1, 0)) - 1
                plsc.store_scatter(wl, [pos], i * L + iota, mask=sel)
                return cnt + plsc.all_reduce_population_count(sel)[0]
            cnt = lax.fori_loop(0, N // L, route, 0)
            wl[pl.ds(sub * NP + cnt, L)] = jnp.full((L,), N, jnp.int32)
            ngrps.append((cnt + (L - 1)) // L)

        for sub in range(NSUB):
            slo = plo + sub * SUBP

            for cp in zdescs:
                cp.wait()

            def grp_body(g, _, slo=slo, sub=sub):
                ids = wl[pl.ds(sub * NP + g * L, L)]
                sfg = plsc.load_gather(sfv, [ids])
                efg = plsc.load_gather(efv, [ids])
                velg = plsc.load_gather(velv, [ids])
                pitg = plsc.load_gather(pitv, [ids])

                s0i = sfg.astype(jnp.int32)
                cfi = efg.astype(jnp.int32)
                ceii = cfi + jnp.where(cfi.astype(jnp.float32) < efg, 1, 0)
                s0g = s0i.astype(jnp.float32)
                e0g = jnp.minimum(ceii.astype(jnp.float32), T_CAP)
                lo16 = jnp.maximum(s0i - W, 0) & ~(L - 1)
                hi16 = jnp.minimum(ceii + (W + 1), T)
                row16 = pitg - slo

                def lane_body(j, _):
                    lane = jnp.zeros((L,), jnp.int32) + j
                    m = iota == j
                    sfb = _take16(sfg, lane)
                    efb = _take16(efg, lane)
                    velb = _take16(velg, lane)
                    s0b = _take16(s0g, lane)
                    e0b = _take16(e0g, lane)
                    loj = jnp.sum(jnp.where(m, lo16, 0))
                    hij = jnp.sum(jnp.where(m, hi16, 0))
                    rowj = jnp.sum(jnp.where(m, row16, 0))
                    ntrip = (jnp.maximum(hij - loj, 0) + (L - 1)) // L

                    def frame_body(kk, _):
                        t0 = loj + kk * L
                        tvi = t0 + iota
                        tvf = tvi.astype(jnp.float32)
                        zon = (tvf - sfb) * INV_SIG
                        on = jnp.exp(-0.5 * zon * zon) * velb
                        box = jnp.where((tvf >= s0b) & (tvf < e0b), 1.0, 0.0)
                        zof = (tvf - efb) * INV_SIG
                        g10 = jnp.where(tvf >= efb,
                                        jnp.exp(-0.5 * zof * zof) * 0.1, 0.0)
                        sus = box + g10
                        velc = sus * velb
                        plsc.addupdate(bon.at[rowj, pl.ds(t0, L)], on)
                        plsc.addupdate(bsus.at[rowj, pl.ds(t0, L)], sus)
                        plsc.addupdate(bvel.at[rowj, pl.ds(t0, L)], velc)
                        return 0

                    lax.fori_loop(0, ntrip, frame_body, 0)
                    return 0
                lax.fori_loop(0, L, lane_body, 0)
                return 0

            lax.fori_loop(0, ngrps[sub], grp_body, 0)

            def clip_grp(g, _, sub=sub):
                ids = wl[pl.ds(sub * NP + g * L, L)]
                sfg = plsc.load_gather(sfv, [ids])
                efg = plsc.load_gather(efv, [ids])
                pitg = plsc.load_gather(pitv, [ids])
                s0i = sfg.astype(jnp.int32)
                cfi = efg.astype(jnp.int32)
                ceii = cfi + jnp.where(cfi.astype(jnp.float32) < efg, 1, 0)
                lo16 = jnp.maximum(s0i - W, 0) & ~(L - 1)
                hi16 = jnp.minimum(ceii + (W + 1), T)
                row16 = pitg - (plo + sub * SUBP)

                def clane(j, _):
                    m = iota == j
                    loj = jnp.sum(jnp.where(m, lo16, 0))
                    hij = jnp.sum(jnp.where(m, hi16, 0))
                    rowj = jnp.sum(jnp.where(m, row16, 0))
                    ntrip = (jnp.maximum(hij - loj, 0) + (L - 1)) // L

                    def cbody(kk, _):
                        sl = pl.ds(loj + kk * L, L)
                        bon[rowj, sl] = jnp.clip(bon[rowj, sl], 0.0, 1.0)
                        sl = pl.ds(loj + kk * L, L)
                        bsus[rowj, sl] = jnp.clip(bsus[rowj, sl], 0.0, 1.0)
                        sl = pl.ds(loj + kk * L, L)
                        bvel[rowj, sl] = jnp.clip(bvel[rowj, sl], 0.0, 1.0)
                        return 0
                    lax.fori_loop(0, ntrip, cbody, 0)
                    return 0
                lax.fori_loop(0, L, clane, 0)
                return 0
            lax.fori_loop(0, ngrps[sub], clip_grp, 0)

            odescs = [
                pltpu.make_async_copy(bon, out_hbm.at[b, pl.ds(slo, SUBP)],
                                      osem),
                pltpu.make_async_copy(bsus, out_hbm.at[b, pl.ds(P + slo, SUBP)],
                                      osem),
                pltpu.make_async_copy(bvel, out_hbm.at[b, pl.ds(2 * P + slo, SUBP)],
                                      osem),
            ]
            for cp in odescs:
                cp.start()
            if sub + 1 < NSUB:
                zdescs = []
                for ch in range(3):
                    odescs[ch].wait()
                    zcp = pltpu.make_async_copy(z_hbm, bufs[ch], zsem)
                    zcp.start()
                    zdescs.append(zcp)
            else:
                for cp in odescs:
                    cp.wait()

    return k(sf, ef, vel, pit, zrows)


def kernel(note_start, note_end, note_vel, note_pitch, dur_sec):
    sr = jnp.float32(SR)
    sf = (note_start * sr).astype(jnp.float32)
    ef = (note_end * sr).astype(jnp.float32)
    vel = note_vel.astype(jnp.float32)
    pit = jnp.clip(note_pitch.astype(jnp.int32), 0, P - 1)
    pad = ((0, 0), (0, L))
    sf = jnp.pad(sf, pad, constant_values=-1e6)
    ef = jnp.pad(ef, pad, constant_values=-1e6)
    vel = jnp.pad(vel, pad, constant_values=0.0)
    pit = jnp.pad(pit, pad, constant_values=-1)
    zrows = jnp.zeros((SUBP, T), jnp.float32)
    return _render_sc(sf, ef, vel, pit, zrows)
